# baseline (speedup 1.0000x reference)
import jax
import jax.numpy as jnp
from jax.experimental import pallas as pl
from jax.experimental.pallas import tpu as pltpu


def _underline_kernel(thr_ref, in_ref, out_ref):
    img = in_ref[0]
    thr = thr_ref[0, 0]
    gray = img[0] * 0.299 + img[1] * 0.587 + img[2] * 0.114
    black = gray < thr
    H, W = gray.shape

    ys2d = jax.lax.broadcasted_iota(jnp.int32, (H, W), 0)
    y1 = jnp.max(jnp.where(black, ys2d, jnp.int32(-1)))

    col_any = jnp.any(black, axis=0, keepdims=True)
    xs = jax.lax.broadcasted_iota(jnp.int32, (1, W), 1)
    x0 = jnp.min(jnp.where(col_any, xs, jnp.int32(W)))
    x1 = jnp.max(jnp.where(col_any, xs, jnp.int32(-1)))

    ys = jax.lax.broadcasted_iota(jnp.int32, (H, 1), 0)
    row_in = ((ys <= y1) & (ys >= y1 - 2)).astype(jnp.float32)
    col_in = ((xs >= x0) & (xs < x1)).astype(jnp.float32)
    keep = 1.0 - row_in * col_in
    out_ref[0] = img * keep[None, :, :]


def kernel(img_tensor, threshold):
    B, C, H, W = img_tensor.shape
    thr = jnp.asarray(threshold, jnp.float32).reshape(1, 1)
    return pl.pallas_call(
        _underline_kernel,
        grid=(B,),
        in_specs=[
            pl.BlockSpec(memory_space=pltpu.SMEM),
            pl.BlockSpec((1, C, H, W), lambda b: (b, 0, 0, 0)),
        ],
        out_specs=pl.BlockSpec((1, C, H, W), lambda b: (b, 0, 0, 0)),
        out_shape=jax.ShapeDtypeStruct((B, C, H, W), img_tensor.dtype),
        compiler_params=pltpu.CompilerParams(
            dimension_semantics=("arbitrary",),
        ),
    )(thr, img_tensor)


# speedup vs baseline: 1.0458x; 1.0458x over previous
import jax
import jax.numpy as jnp
from jax.experimental import pallas as pl
from jax.experimental.pallas import tpu as pltpu


def _underline_kernel(thr_ref, in_ref, out_ref):
    img = in_ref[0]
    thr = thr_ref[0, 0]
    H, W = img.shape[1], img.shape[2]

    out_ref[0] = img

    gray = img[0] * 0.299 + img[1] * 0.587 + img[2] * 0.114
    black = gray < thr

    ys2d = jax.lax.broadcasted_iota(jnp.int32, (H, W), 0)
    y1 = jnp.max(jnp.where(black, ys2d, jnp.int32(-1)))

    col_any = jnp.any(black, axis=0, keepdims=True)
    xs = jax.lax.broadcasted_iota(jnp.int32, (1, W), 1)
    x0 = jnp.min(jnp.where(col_any, xs, jnp.int32(W)))
    x1 = jnp.max(jnp.where(col_any, xs, jnp.int32(-1)))

    start = pl.multiple_of(jnp.clip(((y1 - 2) // 8) * 8, 0, H - 16), 8)
    wys = start + jax.lax.broadcasted_iota(jnp.int32, (16, 1), 0)
    row_in = ((wys <= y1) & (wys >= y1 - 2)).astype(jnp.float32)
    col_in = ((xs >= x0) & (xs < x1)).astype(jnp.float32)
    keep = 1.0 - row_in * col_in
    win = in_ref[0, :, pl.ds(start, 16), :]
    out_ref[0, :, pl.ds(start, 16), :] = win * keep[None, :, :]


def kernel(img_tensor, threshold):
    B, C, H, W = img_tensor.shape
    thr = jnp.asarray(threshold, jnp.float32).reshape(1, 1)
    return pl.pallas_call(
        _underline_kernel,
        grid=(B,),
        in_specs=[
            pl.BlockSpec(memory_space=pltpu.SMEM),
            pl.BlockSpec((1, C, H, W), lambda b: (b, 0, 0, 0)),
        ],
        out_specs=pl.BlockSpec((1, C, H, W), lambda b: (b, 0, 0, 0)),
        out_shape=jax.ShapeDtypeStruct((B, C, H, W), img_tensor.dtype),
        compiler_params=pltpu.CompilerParams(
            dimension_semantics=("arbitrary",),
        ),
    )(thr, img_tensor)
